# baseline (speedup 1.0000x reference)
import jax
import jax.numpy as jnp
from jax.experimental import pallas as pl
from jax.experimental.pallas import tpu as pltpu

_B, _L, _H, _S = 64, 512, 1024, 20
_MAX_SPAN = 10
_K_HOP, _K_ANS = 3, 1
_BB = 4
_R = 2 * _BB

_NEG = -jnp.inf


def _extract(s_mat, e_mat, seps, bst, active, K):
    idx_l = jax.lax.broadcasted_iota(jnp.int32, (_R, _L), 1)
    iota_s = jax.lax.broadcasted_iota(jnp.int32, (_R, _S), 1)
    thresh = s_mat[:, 0:1]
    masked = jnp.where(idx_l >= bst, s_mat, _NEG)

    iota_c = jax.lax.broadcasted_iota(jnp.int32, (_R, 3 * K), 1)
    preds = jnp.zeros((_R, 3 * K), jnp.int32)
    valid = active
    gap = None
    for k in range(K):
        vk = jnp.max(masked, axis=1, keepdims=True)
        sk = jnp.min(jnp.where(masked == vk, idx_l, _L), axis=1, keepdims=True)
        if k + 1 < K:
            masked = jnp.where(idx_l == sk, _NEG, masked)
        cond = (seps > sk) | (seps <= 0)
        jk = jnp.min(jnp.where(cond, iota_s, _S - 1), axis=1, keepdims=True)
        ending = jnp.sum(jnp.where(iota_s == jk, seps, 0), axis=1, keepdims=True)
        ok = (vk > thresh) & (ending > sk)
        valid = valid & ok
        end_cap = jnp.minimum(ending, sk + _MAX_SPAN)
        sel = (idx_l >= sk) & (idx_l < end_cap)
        win = jnp.where(sel, e_mat, _NEG)
        mk = jnp.max(win, axis=1, keepdims=True)
        ek = jnp.min(jnp.where(win == mk, idx_l, _L), axis=1, keepdims=True)
        for c, val in ((0, sk), (1, ek), (2, jk)):
            preds = jnp.where(iota_c == 3 * k + c,
                              jnp.where(valid, val, 0), preds)
        if k == 0:
            gap = jnp.where((vk <= thresh) & active, thresh - vk, 0.0)
    return preds, gap


def _body(x_ref, wT_ref, bT_ref, seps_ref, bst_ref,
          hop_ref, ans_ref, sem_ref, gap_ref):
    x2 = x_ref[...].reshape(_BB * _L, _H)
    ltT = jax.lax.dot_general(
        wT_ref[...], x2, (((1,), (1,)), ((), ())),
        preferred_element_type=jnp.float32) + bT_ref[...]

    def stack_rows(h0, h1):
        return jnp.concatenate(
            [ltT[h:h + 1, s * _L:(s + 1) * _L]
             for h in (h0, h1) for s in range(_BB)], axis=0)

    s_mat = stack_rows(0, 2)
    e_mat = stack_rows(1, 3)

    for s in range(_BB):
        sem_ref[s] = x_ref[2 * s, 0:1, :]

    seps1 = seps_ref[...].reshape(_BB, _S)
    bst1 = bst_ref[...].reshape(_BB, 1)
    seps = jnp.concatenate([seps1, seps1], axis=0)
    bst = jnp.concatenate([bst1, bst1], axis=0)
    active = jnp.min(seps, axis=1, keepdims=True) > 0

    preds, gap = _extract(s_mat, e_mat, seps, bst, active, _K_HOP)
    hop_ref[...] = preds[0:_BB, :].reshape(1, _BB, 3 * _K_HOP)
    ans_ref[...] = preds[_BB:_R, 0:3].reshape(1, _BB, 3)
    gap_ref[...] = gap[_BB:_R, :].reshape(1, _BB, 1)


def kernel(sequence_output, qa_w, qa_b, sep_positions, B_starts,
           hop_start_weights, hop_end_weights, ans_start_weights,
           ans_end_weights):
    del hop_start_weights, hop_end_weights, ans_start_weights, ans_end_weights
    B, L, H = sequence_output.shape
    nb = B // _BB
    xv = sequence_output.reshape(B * 2, L // 2, H)
    wT = qa_w.T
    bT = qa_b.reshape(4, 1)
    seps = sep_positions.reshape(nb, _BB, _S).astype(jnp.int32)
    bst = B_starts.reshape(nb, _BB, 1).astype(jnp.int32)

    grid = (nb,)
    hop, ans, sem3, gap2 = pl.pallas_call(
        _body,
        grid=grid,
        in_specs=[
            pl.BlockSpec((2 * _BB, L // 2, H), lambda i: (i, 0, 0)),
            pl.BlockSpec((4, H), lambda i: (0, 0)),
            pl.BlockSpec((4, 1), lambda i: (0, 0)),
            pl.BlockSpec((1, _BB, _S), lambda i: (i, 0, 0)),
            pl.BlockSpec((1, _BB, 1), lambda i: (i, 0, 0)),
        ],
        out_specs=[
            pl.BlockSpec((1, _BB, 3 * _K_HOP), lambda i: (i, 0, 0)),
            pl.BlockSpec((1, _BB, 3 * _K_ANS), lambda i: (i, 0, 0)),
            pl.BlockSpec((_BB, 1, H), lambda i: (i, 0, 0)),
            pl.BlockSpec((1, _BB, 1), lambda i: (i, 0, 0)),
        ],
        out_shape=[
            jax.ShapeDtypeStruct((nb, _BB, 3 * _K_HOP), jnp.int32),
            jax.ShapeDtypeStruct((nb, _BB, 3 * _K_ANS), jnp.int32),
            jax.ShapeDtypeStruct((B, 1, H), jnp.float32),
            jax.ShapeDtypeStruct((nb, _BB, 1), jnp.float32),
        ],
        compiler_params=pltpu.CompilerParams(
            dimension_semantics=("parallel",),
            vmem_limit_bytes=50 * 1024 * 1024,
        ),
        name="qa_span_extract",
    )(xv, wT, bT, seps, bst)
    return (hop.reshape(B, _K_HOP, 3), ans.reshape(B, _K_ANS, 3),
            sem3.reshape(B, H), gap2.reshape(B))


# speedup vs baseline: 1.1440x; 1.1440x over previous
import jax
import jax.numpy as jnp
from jax.experimental import pallas as pl
from jax.experimental.pallas import tpu as pltpu

_B, _L, _H, _S = 64, 512, 1024, 20
_MAX_SPAN = 10
_K_HOP, _K_ANS = 3, 1
_BB = 8
_R = 2 * _BB

_NEG = -jnp.inf


def _extract(s_mat, e_mat, seps, bst, active, K):
    idx_l = jax.lax.broadcasted_iota(jnp.int32, (_R, _L), 1)
    iota_s = jax.lax.broadcasted_iota(jnp.int32, (_R, _S), 1)
    thresh = s_mat[:, 0:1]
    masked = jnp.where(idx_l >= bst, s_mat, _NEG)

    iota_c = jax.lax.broadcasted_iota(jnp.int32, (_R, 3 * K), 1)
    preds = jnp.zeros((_R, 3 * K), jnp.int32)
    valid = active
    gap = None
    for k in range(K):
        vk = jnp.max(masked, axis=1, keepdims=True)
        sk = jnp.min(jnp.where(masked == vk, idx_l, _L), axis=1, keepdims=True)
        if k + 1 < K:
            masked = jnp.where(idx_l == sk, _NEG, masked)
        cond = (seps > sk) | (seps <= 0)
        jk = jnp.min(jnp.where(cond, iota_s, _S - 1), axis=1, keepdims=True)
        ending = jnp.sum(jnp.where(iota_s == jk, seps, 0), axis=1, keepdims=True)
        ok = (vk > thresh) & (ending > sk)
        valid = valid & ok
        end_cap = jnp.minimum(ending, sk + _MAX_SPAN)
        sel = (idx_l >= sk) & (idx_l < end_cap)
        win = jnp.where(sel, e_mat, _NEG)
        mk = jnp.max(win, axis=1, keepdims=True)
        ek = jnp.min(jnp.where(win == mk, idx_l, _L), axis=1, keepdims=True)
        for c, val in ((0, sk), (1, ek), (2, jk)):
            preds = jnp.where(iota_c == 3 * k + c,
                              jnp.where(valid, val, 0), preds)
        if k == 0:
            gap = jnp.where((vk <= thresh) & active, thresh - vk, 0.0)
    return preds, gap


def _body(x_ref, wT_ref, bT_ref, seps_ref, bst_ref,
          hop_ref, ans_ref, sem_ref, gap_ref):
    x2 = x_ref[...].reshape(_BB * _L, _H)
    ltT = jax.lax.dot_general(
        wT_ref[...], x2, (((1,), (1,)), ((), ())),
        preferred_element_type=jnp.float32) + bT_ref[...]

    def stack_rows(h0, h1):
        return jnp.concatenate(
            [ltT[h:h + 1, s * _L:(s + 1) * _L]
             for h in (h0, h1) for s in range(_BB)], axis=0)

    s_mat = stack_rows(0, 2)
    e_mat = stack_rows(1, 3)

    for s in range(_BB):
        sem_ref[s] = x_ref[s, 0:1, :]

    seps1 = seps_ref[...]
    bst1 = bst_ref[...]
    seps = jnp.concatenate([seps1, seps1], axis=0)
    bst = jnp.concatenate([bst1, bst1], axis=0)
    active = jnp.min(seps, axis=1, keepdims=True) > 0

    preds, gap = _extract(s_mat, e_mat, seps, bst, active, _K_HOP)
    hop_ref[...] = preds[0:_BB, :]
    ans_ref[...] = preds[_BB:_R, 0:3]
    gap_ref[...] = gap[_BB:_R, :]


def kernel(sequence_output, qa_w, qa_b, sep_positions, B_starts,
           hop_start_weights, hop_end_weights, ans_start_weights,
           ans_end_weights):
    del hop_start_weights, hop_end_weights, ans_start_weights, ans_end_weights
    B, L, H = sequence_output.shape
    wT = qa_w.T
    bT = qa_b.reshape(4, 1)
    seps = sep_positions.astype(jnp.int32)
    bst = B_starts.reshape(B, 1).astype(jnp.int32)

    grid = (B // _BB,)
    hop, ans, sem3, gap2 = pl.pallas_call(
        _body,
        grid=grid,
        in_specs=[
            pl.BlockSpec((_BB, L, H), lambda i: (i, 0, 0)),
            pl.BlockSpec((4, H), lambda i: (0, 0)),
            pl.BlockSpec((4, 1), lambda i: (0, 0)),
            pl.BlockSpec((_BB, _S), lambda i: (i, 0)),
            pl.BlockSpec((_BB, 1), lambda i: (i, 0)),
        ],
        out_specs=[
            pl.BlockSpec((_BB, 3 * _K_HOP), lambda i: (i, 0)),
            pl.BlockSpec((_BB, 3 * _K_ANS), lambda i: (i, 0)),
            pl.BlockSpec((_BB, 1, H), lambda i: (i, 0, 0)),
            pl.BlockSpec((_BB, 1), lambda i: (i, 0)),
        ],
        out_shape=[
            jax.ShapeDtypeStruct((B, 3 * _K_HOP), jnp.int32),
            jax.ShapeDtypeStruct((B, 3 * _K_ANS), jnp.int32),
            jax.ShapeDtypeStruct((B, 1, H), jnp.float32),
            jax.ShapeDtypeStruct((B, 1), jnp.float32),
        ],
        compiler_params=pltpu.CompilerParams(
            dimension_semantics=("parallel",),
            vmem_limit_bytes=50 * 1024 * 1024,
        ),
        name="qa_span_extract",
    )(sequence_output, wT, bT, seps, bst)
    return (hop.reshape(B, _K_HOP, 3), ans.reshape(B, _K_ANS, 3),
            sem3.reshape(B, H), gap2.reshape(B))
